# Initial kernel scaffold; baseline (speedup 1.0000x reference)
import functools

import jax
import jax.numpy as jnp
from jax import lax
from jax.experimental import pallas as pl
from jax.experimental.pallas import tpu as pltpu
from jax.experimental.pallas import tpu_sc as plsc

N = 50000
E = 800000
IN = 32
OUT = 32
K = 25
KS0 = 5

NC = 2
NS = 16
NW = NC * NS
EPW = E // NW
C = 200
NCHUNK = EPW // C
ROWS_PT = N // NS
ZROWS = 625
DEG_PT = 5000
DEG_CH = 1000


def _mm_body(x_ref, wf_ref, wr_ref, b_ref, h_ref, root_ref):
    xb = x_ref[...]
    h_ref[...] = jnp.dot(xb, wf_ref[...], preferred_element_type=jnp.float32)
    root_ref[...] = (
        jnp.dot(xb, wr_ref[...], preferred_element_type=jnp.float32) + b_ref[...]
    )


def _matmul(x, wf, wr, bias2d):
    bn = 1000
    grid = (N // bn,)
    return pl.pallas_call(
        _mm_body,
        grid=grid,
        in_specs=[
            pl.BlockSpec((bn, IN), lambda i: (i, 0)),
            pl.BlockSpec((IN, K * OUT), lambda i: (0, 0)),
            pl.BlockSpec((IN, OUT), lambda i: (0, 0)),
            pl.BlockSpec((1, OUT), lambda i: (0, 0)),
        ],
        out_specs=[
            pl.BlockSpec((bn, K * OUT), lambda i: (i, 0)),
            pl.BlockSpec((bn, OUT), lambda i: (i, 0)),
        ],
        out_shape=[
            jax.ShapeDtypeStruct((N, K * OUT), jnp.float32),
            jax.ShapeDtypeStruct((N, OUT), jnp.float32),
        ],
    )(x, wf, wr, bias2d)


def _prep_body(ps_ref, ec_ref, basis_ref, gidx_ref):
    p0 = ps_ref[0:1, :]
    p1 = ps_ref[1:2, :]
    v0 = p0 * jnp.float32(KS0 - 1)
    v1 = p1 * jnp.float32(KS0 - 1)
    l0 = jnp.floor(v0)
    l1 = jnp.floor(v1)
    f0 = v0 - l0
    f1 = v1 - l1
    l0i = l0.astype(jnp.int32)
    l1i = l1.astype(jnp.int32)
    i0a = jnp.clip(l0i, 0, KS0 - 1)
    i0b = jnp.clip(l0i + 1, 0, KS0 - 1)
    i1a = jnp.clip(l1i, 0, KS0 - 1)
    i1b = jnp.clip(l1i + 1, 0, KS0 - 1)
    base = ec_ref[1:2, :] * K
    basis_ref[0:1, :] = (1.0 - f0) * (1.0 - f1)
    basis_ref[1:2, :] = f0 * (1.0 - f1)
    basis_ref[2:3, :] = (1.0 - f0) * f1
    basis_ref[3:4, :] = f0 * f1
    gidx_ref[0:1, :] = base + i0a + KS0 * i1a
    gidx_ref[1:2, :] = base + i0b + KS0 * i1a
    gidx_ref[2:3, :] = base + i0a + KS0 * i1b
    gidx_ref[3:4, :] = base + i0b + KS0 * i1b


def _edge_prep(pseudo_t, edge_index):
    be = 8000
    grid = (E // be,)
    return pl.pallas_call(
        _prep_body,
        grid=grid,
        in_specs=[
            pl.BlockSpec((2, be), lambda i: (0, i)),
            pl.BlockSpec((2, be), lambda i: (0, i)),
        ],
        out_specs=[
            pl.BlockSpec((4, be), lambda i: (0, i)),
            pl.BlockSpec((4, be), lambda i: (0, i)),
        ],
        out_shape=[
            jax.ShapeDtypeStruct((4, E), jnp.float32),
            jax.ShapeDtypeStruct((4, E), jnp.int32),
        ],
    )(pseudo_t, edge_index)


def _sc_body(h_hbm, gidx_hbm, basis_hbm, row_hbm, acc_hbm, deg_hbm,
             gidx_v, basis_v, row_v, rows_v, msg_v, ones_v, zbuf, zdeg,
             acc_sh, deg_sh, sem):
    cid = lax.axis_index("c")
    sid = lax.axis_index("s")
    wid = cid * NS + sid

    zvec = jnp.zeros((16,), jnp.float32)

    def _zb(i, carry):
        zbuf[i, pl.ds(0, 16)] = zvec
        zbuf[i, pl.ds(16, 16)] = zvec
        return carry

    lax.fori_loop(0, ZROWS, _zb, 0)

    def _zd(i, carry):
        zdeg[pl.ds(i * 16, 16)] = zvec
        return carry

    lax.fori_loop(0, 63, _zd, 0)

    def _zo(i, carry):
        ones_v[pl.ds(i * 16, 16)] = zvec + 1.0
        return carry

    lax.fori_loop(0, 13, _zo, 0)

    r0 = sid * ROWS_PT
    for i in range(ROWS_PT // ZROWS):
        pltpu.sync_copy(zbuf, acc_sh.at[pl.ds(r0 + i * ZROWS, ZROWS), :])

    @pl.when(sid < 10)
    def _():
        d0 = sid * DEG_PT
        for i in range(DEG_PT // DEG_CH):
            pltpu.sync_copy(zdeg.at[pl.ds(0, DEG_CH)],
                            deg_sh.at[pl.ds(d0 + i * DEG_CH, DEG_CH)])

    plsc.subcore_barrier()

    ebase = wid * EPW

    def _chunk(b, carry):
        base = ebase + b * C
        pltpu.sync_copy(gidx_hbm.at[:, pl.ds(base, C)], gidx_v)
        pltpu.sync_copy(basis_hbm.at[:, pl.ds(base, C)], basis_v)
        pltpu.sync_copy(row_hbm.at[pl.ds(base, C)], row_v)
        cps = [pltpu.async_copy(h_hbm.at[gidx_v.at[s]], rows_v.at[s], sem)
               for s in range(4)]
        for cp in cps:
            cp.wait()

        def _edge(e, ecarry):
            b0 = basis_v[0, e]
            b1 = basis_v[1, e]
            b2 = basis_v[2, e]
            b3 = basis_v[3, e]
            lo = (b0 * rows_v[0, e, pl.ds(0, 16)]
                  + b1 * rows_v[1, e, pl.ds(0, 16)]
                  + b2 * rows_v[2, e, pl.ds(0, 16)]
                  + b3 * rows_v[3, e, pl.ds(0, 16)])
            hi = (b0 * rows_v[0, e, pl.ds(16, 16)]
                  + b1 * rows_v[1, e, pl.ds(16, 16)]
                  + b2 * rows_v[2, e, pl.ds(16, 16)]
                  + b3 * rows_v[3, e, pl.ds(16, 16)])
            msg_v[e, pl.ds(0, 16)] = lo
            msg_v[e, pl.ds(16, 16)] = hi
            return ecarry

        lax.fori_loop(0, C, _edge, 0)

        pltpu.sync_copy(msg_v, acc_sh.at[row_v], add=True)
        pltpu.sync_copy(ones_v.at[pl.ds(0, C)], deg_sh.at[row_v], add=True)
        return carry

    lax.fori_loop(0, NCHUNK, _chunk, 0)

    plsc.subcore_barrier()

    for i in range(ROWS_PT // ZROWS):
        pltpu.sync_copy(acc_sh.at[pl.ds(r0 + i * ZROWS, ZROWS), :], zbuf)
        pltpu.sync_copy(zbuf, acc_hbm.at[cid, pl.ds(r0 + i * ZROWS, ZROWS), :])

    @pl.when(sid < 10)
    def _():
        d0 = sid * DEG_PT
        for i in range(DEG_PT // DEG_CH):
            pltpu.sync_copy(deg_sh.at[pl.ds(d0 + i * DEG_CH, DEG_CH)],
                            zdeg.at[pl.ds(0, DEG_CH)])
            pltpu.sync_copy(zdeg.at[pl.ds(0, DEG_CH)],
                            deg_hbm.at[cid, pl.ds(d0 + i * DEG_CH, DEG_CH)])


def _sc_scatter(h2, gidx, basis, row):
    mesh = plsc.VectorSubcoreMesh(core_axis_name="c", subcore_axis_name="s")
    fn = pl.kernel(
        _sc_body,
        out_type=[
            jax.ShapeDtypeStruct((NC, N, OUT), jnp.float32),
            jax.ShapeDtypeStruct((NC, N), jnp.float32),
        ],
        mesh=mesh,
        scratch_types=[
            pltpu.VMEM((4, C), jnp.int32),
            pltpu.VMEM((4, C), jnp.float32),
            pltpu.VMEM((C,), jnp.int32),
            pltpu.VMEM((4, C, OUT), jnp.float32),
            pltpu.VMEM((C, OUT), jnp.float32),
            pltpu.VMEM((208,), jnp.float32),
            pltpu.VMEM((ZROWS, OUT), jnp.float32),
            pltpu.VMEM((1008,), jnp.float32),
            pltpu.VMEM_SHARED((N, OUT), jnp.float32),
            pltpu.VMEM_SHARED((N,), jnp.float32),
            pltpu.SemaphoreType.DMA,
        ],
    )
    return fn(h2, gidx, basis, row)


def _comb_body(acc_ref, deg_ref, root_ref, o_ref):
    a = acc_ref[0] + acc_ref[1]
    d = jnp.maximum(deg_ref[0] + deg_ref[1], 1.0)
    o_ref[...] = a / d + root_ref[...]


def _combine(acc, deg3, root):
    bn = 2000
    grid = (N // bn,)
    return pl.pallas_call(
        _comb_body,
        grid=grid,
        in_specs=[
            pl.BlockSpec((NC, bn, OUT), lambda i: (0, i, 0)),
            pl.BlockSpec((NC, bn, 1), lambda i: (0, i, 0)),
            pl.BlockSpec((bn, OUT), lambda i: (i, 0)),
        ],
        out_specs=pl.BlockSpec((bn, OUT), lambda i: (i, 0)),
        out_shape=jax.ShapeDtypeStruct((N, OUT), jnp.float32),
    )(acc, deg3, root)


@jax.jit
def kernel(x, edge_index, pseudo, weight, bias):
    wf = jnp.transpose(weight[:K], (1, 0, 2)).reshape(IN, K * OUT)
    wr = weight[K]
    h, root = _matmul(x, wf, wr, bias.reshape(1, OUT))
    h2 = h.reshape(N * K, OUT)
    basis, gidx = _edge_prep(jnp.transpose(pseudo), edge_index)
    row = edge_index[0]
    acc, deg = _sc_scatter(h2, gidx, basis, row)
    out = _combine(acc, deg.reshape(NC, N, 1), root)
    return out


# speedup vs baseline: 3.8965x; 3.8965x over previous
import functools

import jax
import jax.numpy as jnp
from jax import lax
from jax.experimental import pallas as pl
from jax.experimental.pallas import tpu as pltpu
from jax.experimental.pallas import tpu_sc as plsc

N = 50000
E = 800000
E2 = 802816
PAD = E2 - E
IN = 32
OUT = 32
K = 25
KS0 = 5

NC = 2
NS = 16
NW = NC * NS
C = 256
CQ = C // 128
EPS = E2 // NS
NCHUNK = EPS // C
HALF = 16
RCH = 400
NFULL = (N // RCH) // NS
NCHREM = (N // RCH) - NFULL * NS


def _mm_body(x_ref, wf_ref, wr_ref, b_ref, h_ref, root_ref):
    xb = x_ref[...]
    h_ref[...] = jnp.dot(xb, wf_ref[...], preferred_element_type=jnp.float32)
    root_ref[...] = (
        jnp.dot(xb, wr_ref[...], preferred_element_type=jnp.float32) + b_ref[...]
    )


def _matmul(x, wf, wr, bias2d):
    bn = 1000
    grid = (N // bn,)
    return pl.pallas_call(
        _mm_body,
        grid=grid,
        in_specs=[
            pl.BlockSpec((bn, IN), lambda i: (i, 0)),
            pl.BlockSpec((IN, K * OUT), lambda i: (0, 0)),
            pl.BlockSpec((IN, OUT), lambda i: (0, 0)),
            pl.BlockSpec((1, OUT), lambda i: (0, 0)),
        ],
        out_specs=[
            pl.BlockSpec((bn, K * OUT), lambda i: (i, 0)),
            pl.BlockSpec((bn, OUT), lambda i: (i, 0)),
        ],
        out_shape=[
            jax.ShapeDtypeStruct((N, K * OUT), jnp.float32),
            jax.ShapeDtypeStruct((N, OUT), jnp.float32),
        ],
    )(x, wf, wr, bias2d)


def _prep_body(ps_ref, ec_ref, basis_ref, gidxa_ref, gidxb_ref):
    p0 = ps_ref[0:1, :]
    p1 = ps_ref[1:2, :]
    v0 = p0 * jnp.float32(KS0 - 1)
    v1 = p1 * jnp.float32(KS0 - 1)
    l0 = jnp.floor(v0)
    l1 = jnp.floor(v1)
    f0 = v0 - l0
    f1 = v1 - l1
    l0i = l0.astype(jnp.int32)
    l1i = l1.astype(jnp.int32)
    i0a = jnp.clip(l0i, 0, KS0 - 1)
    i0b = jnp.clip(l0i + 1, 0, KS0 - 1)
    i1a = jnp.clip(l1i, 0, KS0 - 1)
    i1b = jnp.clip(l1i + 1, 0, KS0 - 1)
    base = ec_ref[1:2, :] * K
    basis_ref[0:1, :] = (1.0 - f0) * (1.0 - f1)
    basis_ref[1:2, :] = f0 * (1.0 - f1)
    basis_ref[2:3, :] = (1.0 - f0) * f1
    basis_ref[3:4, :] = f0 * f1
    ga0 = (base + i0a + KS0 * i1a) * 2
    ga1 = (base + i0b + KS0 * i1a) * 2
    ga2 = (base + i0a + KS0 * i1b) * 2
    ga3 = (base + i0b + KS0 * i1b) * 2
    gidxa_ref[0:1, :] = ga0
    gidxa_ref[1:2, :] = ga1
    gidxa_ref[2:3, :] = ga2
    gidxa_ref[3:4, :] = ga3
    gidxb_ref[0:1, :] = ga0 + 1
    gidxb_ref[1:2, :] = ga1 + 1
    gidxb_ref[2:3, :] = ga2 + 1
    gidxb_ref[3:4, :] = ga3 + 1


def _edge_prep(pseudo_t, edge_index):
    be = 6272
    grid = (E2 // be,)
    return pl.pallas_call(
        _prep_body,
        grid=grid,
        in_specs=[
            pl.BlockSpec((2, be), lambda i: (0, i)),
            pl.BlockSpec((2, be), lambda i: (0, i)),
        ],
        out_specs=[
            pl.BlockSpec((4, be), lambda i: (0, i)),
            pl.BlockSpec((4, be), lambda i: (0, i)),
            pl.BlockSpec((4, be), lambda i: (0, i)),
        ],
        out_shape=[
            jax.ShapeDtypeStruct((4, E2), jnp.float32),
            jax.ShapeDtypeStruct((4, E2), jnp.int32),
            jax.ShapeDtypeStruct((4, E2), jnp.int32),
        ],
    )(pseudo_t, edge_index)


def _sc_body(h_hbm, gidxa_hbm, gidxb_hbm, basis_hbm, row_hbm,
             acc_hbm, deg_hbm,
             gidx_v, basis_v, row_v, rows_v, msg_v, hist_v, zbuf,
             acc_sh, sem):
    cid = lax.axis_index("c")
    sid = lax.axis_index("s")

    zvec = jnp.zeros((16,), jnp.float32)
    ones16 = zvec + 1.0

    def _zb(i, carry):
        zbuf[i, :] = zvec
        return carry

    lax.fori_loop(0, RCH, _zb, 0)

    def _zh(i, carry):
        hist_v[pl.ds(i * 16, 16)] = zvec
        return carry

    lax.fori_loop(0, (EPS + 16) // 16, _zh, 0)

    def _zero_chunk(ch):
        r = pl.multiple_of(ch * RCH, 8)
        pltpu.sync_copy(zbuf, acc_sh.at[pl.ds(r, RCH), :])

    for t in range(NFULL):
        _zero_chunk(t * NS + sid)

    @pl.when(sid < NCHREM)
    def _():
        _zero_chunk(NFULL * NS + sid)

    plsc.subcore_barrier()

    ebase = sid * EPS

    def _chunk(b, carry):
        base = ebase + b * C

        @pl.when(cid == 0)
        def _():
            pltpu.sync_copy(gidxa_hbm.at[:, pl.ds(base, C)], gidx_v)

        @pl.when(cid == 1)
        def _():
            pltpu.sync_copy(gidxb_hbm.at[:, pl.ds(base, C)], gidx_v)

        pltpu.sync_copy(basis_hbm.at[:, pl.ds(base, C)], basis_v)
        rb = sid * (EPS // 128) + b * CQ
        pltpu.sync_copy(row_hbm.at[pl.ds(rb, CQ), :], row_v)
        cps = [pltpu.async_copy(
                   h_hbm.at[gidx_v.at[s, pl.ds(q * 128, 128)]],
                   rows_v.at[s, pl.ds(q * 128, 128), :], sem)
               for s in range(4) for q in range(CQ)]
        for cp in cps:
            cp.wait()

        def _grp(g, gcarry):
            e0 = g * 16
            bs = [basis_v[s, pl.ds(e0, 16)] for s in range(4)]
            ix = row_v[g // 8, pl.ds((g % 8) * 16, 16)]
            plsc.addupdate_scatter(hist_v, [ix], ones16)
            for j in range(16):
                e = e0 + j
                msg_v[e, :] = (bs[0][j] * rows_v[0, e, :]
                               + bs[1][j] * rows_v[1, e, :]
                               + bs[2][j] * rows_v[2, e, :]
                               + bs[3][j] * rows_v[3, e, :])
            return gcarry

        lax.fori_loop(0, C // 16, _grp, 0)

        for q in range(CQ):
            pltpu.sync_copy(msg_v.at[pl.ds(q * 128, 128), :],
                            acc_sh.at[row_v.at[q]], add=True)
        return carry

    lax.fori_loop(0, NCHUNK, _chunk, 0)

    plsc.subcore_barrier()

    def _out_chunk(ch):
        r = pl.multiple_of(ch * RCH, 8)
        pltpu.sync_copy(acc_sh.at[pl.ds(r, RCH), :], zbuf)
        pltpu.sync_copy(zbuf, acc_hbm.at[cid, pl.ds(r, RCH), :])

    for t in range(NFULL):
        _out_chunk(t * NS + sid)

    @pl.when(sid < NCHREM)
    def _():
        _out_chunk(NFULL * NS + sid)

    def _dout(i, carry):
        r = pl.multiple_of(i * RCH, 8)
        pltpu.sync_copy(hist_v.at[pl.ds(r, RCH)],
                        deg_hbm.at[cid, sid, pl.ds(r, RCH)])
        return carry

    lax.fori_loop(0, N // RCH, _dout, 0)


def _sc_scatter(h2, gidxa, gidxb, basis, row):
    mesh = plsc.VectorSubcoreMesh(core_axis_name="c", subcore_axis_name="s")
    fn = pl.kernel(
        _sc_body,
        out_type=[
            jax.ShapeDtypeStruct((NC, N, HALF), jnp.float32),
            jax.ShapeDtypeStruct((NC, NS, N), jnp.float32),
        ],
        mesh=mesh,
        compiler_params=pltpu.CompilerParams(use_tc_tiling_on_sc=False,
                                             needs_layout_passes=False),
        scratch_types=[
            pltpu.VMEM((4, C), jnp.int32),
            pltpu.VMEM((4, C), jnp.float32),
            pltpu.VMEM((CQ, 128), jnp.int32),
            pltpu.VMEM((4, C, HALF), jnp.float32),
            pltpu.VMEM((C, HALF), jnp.float32),
            pltpu.VMEM((EPS + 16,), jnp.float32),
            pltpu.VMEM((RCH, HALF), jnp.float32),
            pltpu.VMEM_SHARED((N + 8, HALF), jnp.float32),
            pltpu.SemaphoreType.DMA,
        ],
    )
    return fn(h2, gidxa, gidxb, basis, row)


def _comb_body(acc_ref, deg_ref, root_ref, o_ref):
    a = jnp.concatenate([acc_ref[0], acc_ref[1]], axis=1)
    d = 0.5 * jnp.sum(deg_ref[...], axis=1, keepdims=True)
    o_ref[...] = a / jnp.maximum(d, 1.0) + root_ref[...]


def _combine(acc, degt, root):
    bn = 2000
    grid = (N // bn,)
    return pl.pallas_call(
        _comb_body,
        grid=grid,
        in_specs=[
            pl.BlockSpec((NC, bn, HALF), lambda i: (0, i, 0)),
            pl.BlockSpec((bn, NW), lambda i: (i, 0)),
            pl.BlockSpec((bn, OUT), lambda i: (i, 0)),
        ],
        out_specs=pl.BlockSpec((bn, OUT), lambda i: (i, 0)),
        out_shape=jax.ShapeDtypeStruct((N, OUT), jnp.float32),
    )(acc, degt, root)


@jax.jit
def kernel(x, edge_index, pseudo, weight, bias):
    wf = jnp.transpose(weight[:K], (1, 0, 2)).reshape(IN, K * OUT)
    wr = weight[K]
    h, root = _matmul(x, wf, wr, bias.reshape(1, OUT))
    h2 = h.reshape(N * K * 2, HALF)
    ec2 = jnp.pad(edge_index, ((0, 0), (0, PAD)))
    ps2 = jnp.pad(jnp.transpose(pseudo), ((0, 0), (0, PAD)))
    basis, gidxa, gidxb = _edge_prep(ps2, ec2)
    row = jnp.concatenate(
        [edge_index[0], jnp.full((PAD,), N, jnp.int32)]).reshape(E2 // 128, 128)
    acc, deg = _sc_scatter(h2, gidxa, gidxb, basis, row)
    out = _combine(acc, jnp.transpose(deg.reshape(NW, N)), root)
    return out
